# Initial kernel scaffold; baseline (speedup 1.0000x reference)
import jax
import jax.numpy as jnp
from jax.experimental import pallas as pl


def kernel(img):
    raise NotImplementedError("write your pallas kernel here")


# speedup vs baseline: 5.0999x; 5.0999x over previous
import functools

import jax
import jax.numpy as jnp
from jax import lax
from jax.experimental import pallas as pl
from jax.experimental.pallas import tpu as pltpu
from jax.experimental.pallas import tpu_sc as plsc

B, H, W, C = 4, 224, 224, 96
H2, W2 = 2 * H, 2 * W
NC, NS = 2, 16
NW = NC * NS
XHW = W // 2
XOW = 2 * XHW
NCHUNKS = B * H * 2
NCH = NCHUNKS // NW
CREG = C // 16


def _sc_body(in_hbm, out_hbm, ib0, ib1, xd0, xd1, gs0, gs1, ss0, ss1):
    wid = lax.axis_index("s") * NC + lax.axis_index("c")
    ibufs, xbufs = (ib0, ib1), (xd0, xd1)
    gsems, ssems = (gs0, gs1), (ss0, ss1)

    def locate(c):
        g = wid * NCH + c
        b = g // (2 * H)
        rem = g - b * (2 * H)
        yi = rem // 2
        xh = rem - yi * 2
        return b, yi, xh

    def gather(c, slot):
        b, yi, xh = locate(c)
        return pltpu.make_async_copy(
            in_hbm.at[b, yi, pl.ds(xh * XHW, XHW)], ibufs[slot], gsems[slot])

    def scatter(c, slot, a):
        b, yi, xh = locate(c)
        return pltpu.make_async_copy(
            xbufs[slot], out_hbm.at[b, 2 * yi + a, pl.ds(xh * XOW, XOW)],
            ssems[slot])

    def interleave(slot):
        ib, xd = ibufs[slot], xbufs[slot]

        def row(r, carry):
            for k in range(CREG):
                v = ib[r, pl.ds(k * 16, 16)]
                xd[2 * r, pl.ds(k * 16, 16)] = v
                xd[2 * r + 1, pl.ds(k * 16, 16)] = v
            return carry

        lax.fori_loop(0, XHW, row, 0)

    def proc(c, slot, drain, prefetch):
        gather(c, slot).wait()
        if drain:
            scatter(c - 2, slot, 0).wait()
            scatter(c - 2, slot, 1).wait()
        interleave(slot)
        scatter(c, slot, 0).start()
        scatter(c, slot, 1).start()
        if prefetch:
            gather(c + 2, slot).start()

    gather(0, 0).start()
    gather(1, 1).start()
    proc(0, 0, drain=False, prefetch=True)
    proc(1, 1, drain=False, prefetch=True)

    def body(i, carry):
        proc(2 + 2 * i, 0, drain=True, prefetch=True)
        proc(3 + 2 * i, 1, drain=True, prefetch=True)
        return carry

    lax.fori_loop(0, (NCH - 4) // 2, body, 0)
    proc(NCH - 2, 0, drain=True, prefetch=False)
    proc(NCH - 1, 1, drain=True, prefetch=False)
    for slot, c in ((0, NCH - 2), (1, NCH - 1)):
        scatter(c, slot, 0).wait()
        scatter(c, slot, 1).wait()


_sc_upsample = functools.partial(
    pl.kernel,
    mesh=plsc.VectorSubcoreMesh(core_axis_name="c", subcore_axis_name="s"),
    out_type=jax.ShapeDtypeStruct((B, H2, W2, C), jnp.float32),
    scratch_types=[
        pltpu.VMEM((XHW, C), jnp.float32),
        pltpu.VMEM((XHW, C), jnp.float32),
        pltpu.VMEM((XOW, C), jnp.float32),
        pltpu.VMEM((XOW, C), jnp.float32),
        pltpu.SemaphoreType.DMA,
        pltpu.SemaphoreType.DMA,
        pltpu.SemaphoreType.DMA,
        pltpu.SemaphoreType.DMA,
    ],
)(_sc_body)


@jax.jit
def kernel(img):
    return _sc_upsample(img)


# speedup vs baseline: 9.9539x; 1.9518x over previous
import functools

import jax
import jax.numpy as jnp
from jax import lax
from jax.experimental import pallas as pl
from jax.experimental.pallas import tpu as pltpu
from jax.experimental.pallas import tpu_sc as plsc

B, H, W, C = 4, 224, 224, 96
H2, W2 = 2 * H, 2 * W
NC, NS = 2, 16
NW = NC * NS
CH = C // 2
NCHUNKS = B * H * 2
NCH = NCHUNKS // NW
NJ = W2 // 16


def _sc_body(in_hbm, out_hbm, ib0, ib1, xd0, xd1, gs0, gs1, ss0, ss1):
    wid = lax.axis_index("s") * NC + lax.axis_index("c")
    ibufs, xbufs = (ib0, ib1), (xd0, xd1)
    gsems, ssems = (gs0, gs1), (ss0, ss1)
    def locate(t):
        g = wid * NCH + t
        b = g // (2 * H)
        rem = g - b * (2 * H)
        yi = rem // 2
        ch = rem - yi * 2
        return b, yi, ch

    def gather(t, slot):
        b, yi, ch = locate(t)
        return pltpu.make_async_copy(
            in_hbm.at[b, yi, pl.ds(ch * CH, CH)], ibufs[slot], gsems[slot])

    def scatter(t, slot, a):
        b, yi, ch = locate(t)
        return pltpu.make_async_copy(
            xbufs[slot], out_hbm.at[b, 2 * yi + a, pl.ds(ch * CH, CH)],
            ssems[slot])

    def interleave(slot):
        ib, xd = ibufs[slot], xbufs[slot]

        def crow(ci, carry):
            it = lax.broadcasted_iota(jnp.int32, (16,), 0)
            half = lax.shift_right_logical(it, 1)
            rowi = jnp.full((16,), ci, jnp.int32)
            for j in range(NJ):
                v = plsc.load_gather(ib, [rowi, half + 8 * j])
                xd[ci, pl.ds(16 * j, 16)] = v
            return carry

        lax.fori_loop(0, CH, crow, 0)

    def proc(t, slot, drain, prefetch):
        gather(t, slot).wait()
        if drain:
            scatter(t - 2, slot, 0).wait()
            scatter(t - 2, slot, 1).wait()
        interleave(slot)
        scatter(t, slot, 0).start()
        scatter(t, slot, 1).start()
        if prefetch:
            gather(t + 2, slot).start()

    gather(0, 0).start()
    gather(1, 1).start()
    proc(0, 0, drain=False, prefetch=True)
    proc(1, 1, drain=False, prefetch=True)

    def body(i, carry):
        proc(2 + 2 * i, 0, drain=True, prefetch=True)
        proc(3 + 2 * i, 1, drain=True, prefetch=True)
        return carry

    lax.fori_loop(0, (NCH - 4) // 2, body, 0)
    proc(NCH - 2, 0, drain=True, prefetch=False)
    proc(NCH - 1, 1, drain=True, prefetch=False)
    for slot, t in ((0, NCH - 2), (1, NCH - 1)):
        scatter(t, slot, 0).wait()
        scatter(t, slot, 1).wait()


_sc_upsample = functools.partial(
    pl.kernel,
    mesh=plsc.VectorSubcoreMesh(core_axis_name="c", subcore_axis_name="s"),
    out_type=jax.ShapeDtypeStruct((B, H2, C, W2), jnp.float32),
    scratch_types=[
        pltpu.VMEM((CH, W), jnp.float32),
        pltpu.VMEM((CH, W), jnp.float32),
        pltpu.VMEM((CH, W2), jnp.float32),
        pltpu.VMEM((CH, W2), jnp.float32),
        pltpu.SemaphoreType.DMA,
        pltpu.SemaphoreType.DMA,
        pltpu.SemaphoreType.DMA,
        pltpu.SemaphoreType.DMA,
    ],
    compiler_params=pltpu.CompilerParams(needs_layout_passes=False),
)(_sc_body)


@jax.jit
def kernel(img):
    imgt = img.transpose(0, 1, 3, 2)
    outt = _sc_upsample(imgt)
    return outt.transpose(0, 1, 3, 2)


# speedup vs baseline: 14.9664x; 1.5036x over previous
import functools

import jax
import jax.numpy as jnp
from jax import lax
from jax.experimental import pallas as pl
from jax.experimental.pallas import tpu as pltpu
from jax.experimental.pallas import tpu_sc as plsc

B, H, W, C = 4, 224, 224, 96
H2, W2 = 2 * H, 2 * W
NC, NS = 2, 16
NW = NC * NS
CH = C // 2
NCHUNKS = B * H * 2
NCH = NCHUNKS // NW
NJ = W2 // 16


def _sc_body(in_hbm, out_hbm, ib0, ib1, xd0, xd1, gs0, gs1, ss0, ss1):
    wid = lax.axis_index("s") * NC + lax.axis_index("c")
    ibufs, xbufs = (ib0, ib1), (xd0, xd1)
    gsems, ssems = (gs0, gs1), (ss0, ss1)
    def locate(t):
        g = wid * NCH + t
        b = g // (2 * H)
        rem = g - b * (2 * H)
        yi = rem // 2
        ch = rem - yi * 2
        return b, yi, ch

    def gather(t, slot):
        b, yi, ch = locate(t)
        return pltpu.make_async_copy(
            in_hbm.at[b, yi, pl.ds(ch * CH, CH)], ibufs[slot], gsems[slot])

    def scatter(t, slot, a):
        b, yi, ch = locate(t)
        return pltpu.make_async_copy(
            xbufs[slot], out_hbm.at[b, 2 * yi + a, pl.ds(ch * CH, CH)],
            ssems[slot])

    def interleave(slot):
        ib, xd = ibufs[slot], xbufs[slot]

        def crow(ci, carry):
            it = lax.broadcasted_iota(jnp.int32, (16,), 0)
            h0 = lax.shift_right_logical(it, 1)
            h1 = h0 + 8
            for k in range(W // 16):
                v = ib[ci, pl.ds(16 * k, 16)]
                xd[ci, pl.ds(32 * k, 16)] = jnp.take_along_axis(v, h0, axis=0)
                xd[ci, pl.ds(32 * k + 16, 16)] = jnp.take_along_axis(v, h1, axis=0)
            return carry

        lax.fori_loop(0, CH, crow, 0)

    def proc(t, slot, drain, prefetch):
        gather(t, slot).wait()
        if drain:
            scatter(t - 2, slot, 0).wait()
            scatter(t - 2, slot, 1).wait()
        interleave(slot)
        scatter(t, slot, 0).start()
        scatter(t, slot, 1).start()
        if prefetch:
            gather(t + 2, slot).start()

    gather(0, 0).start()
    gather(1, 1).start()
    proc(0, 0, drain=False, prefetch=True)
    proc(1, 1, drain=False, prefetch=True)

    def body(i, carry):
        proc(2 + 2 * i, 0, drain=True, prefetch=True)
        proc(3 + 2 * i, 1, drain=True, prefetch=True)
        return carry

    lax.fori_loop(0, (NCH - 4) // 2, body, 0)
    proc(NCH - 2, 0, drain=True, prefetch=False)
    proc(NCH - 1, 1, drain=True, prefetch=False)
    for slot, t in ((0, NCH - 2), (1, NCH - 1)):
        scatter(t, slot, 0).wait()
        scatter(t, slot, 1).wait()


_sc_upsample = functools.partial(
    pl.kernel,
    mesh=plsc.VectorSubcoreMesh(core_axis_name="c", subcore_axis_name="s"),
    out_type=jax.ShapeDtypeStruct((B, H2, C, W2), jnp.float32),
    scratch_types=[
        pltpu.VMEM((CH, W), jnp.float32),
        pltpu.VMEM((CH, W), jnp.float32),
        pltpu.VMEM((CH, W2), jnp.float32),
        pltpu.VMEM((CH, W2), jnp.float32),
        pltpu.SemaphoreType.DMA,
        pltpu.SemaphoreType.DMA,
        pltpu.SemaphoreType.DMA,
        pltpu.SemaphoreType.DMA,
    ],
    compiler_params=pltpu.CompilerParams(needs_layout_passes=False),
)(_sc_body)


@jax.jit
def kernel(img):
    imgt = img.transpose(0, 1, 3, 2)
    outt = _sc_upsample(imgt)
    return outt.transpose(0, 1, 3, 2)


# speedup vs baseline: 17.7877x; 1.1885x over previous
import functools

import jax
import jax.numpy as jnp
from jax import lax
from jax.experimental import pallas as pl
from jax.experimental.pallas import tpu as pltpu
from jax.experimental.pallas import tpu_sc as plsc

B, H, W, C = 4, 224, 224, 96
H2, W2 = 2 * H, 2 * W
NC, NS = 2, 16
NW = NC * NS
CH = C // 2
NCHUNKS = B * H * 2
NCH = NCHUNKS // NW
NJ = W2 // 16


def _sc_body(in_hbm, out_hbm, ib0, ib1, xd0, xd1, gs0, gs1, ss0, ss1):
    wid = lax.axis_index("s") * NC + lax.axis_index("c")
    ibufs, xbufs = (ib0, ib1), (xd0, xd1)
    gsems, ssems = (gs0, gs1), (ss0, ss1)
    def locate(t):
        g = wid * NCH + t
        b = g // (2 * H)
        rem = g - b * (2 * H)
        yi = rem // 2
        ch = rem - yi * 2
        return b, yi, ch

    def gather(t, slot):
        b, yi, ch = locate(t)
        return pltpu.make_async_copy(
            in_hbm.at[b, yi, pl.ds(ch * CH, CH)], ibufs[slot], gsems[slot])

    def scatter(t, slot, a):
        b, yi, ch = locate(t)
        return pltpu.make_async_copy(
            xbufs[slot], out_hbm.at[b, 2 * yi + a, pl.ds(ch * CH, CH)],
            ssems[slot])

    def interleave(slot):
        ib, xd = ibufs[slot], xbufs[slot]

        @plsc.parallel_loop(0, CH, 1, unroll=8)
        def crow(ci):
            it = lax.broadcasted_iota(jnp.int32, (16,), 0)
            h0 = lax.shift_right_logical(it, 1)
            h1 = h0 + 8
            for k in range(W // 16):
                v = ib[ci, pl.ds(16 * k, 16)]
                xd[ci, pl.ds(32 * k, 16)] = jnp.take_along_axis(v, h0, axis=0)
                xd[ci, pl.ds(32 * k + 16, 16)] = jnp.take_along_axis(v, h1, axis=0)

    def proc(t, slot, drain, prefetch):
        gather(t, slot).wait()
        if drain:
            scatter(t - 2, slot, 0).wait()
            scatter(t - 2, slot, 1).wait()
        interleave(slot)
        scatter(t, slot, 0).start()
        scatter(t, slot, 1).start()
        if prefetch:
            gather(t + 2, slot).start()

    gather(0, 0).start()
    gather(1, 1).start()
    proc(0, 0, drain=False, prefetch=True)
    proc(1, 1, drain=False, prefetch=True)

    def body(i, carry):
        proc(2 + 2 * i, 0, drain=True, prefetch=True)
        proc(3 + 2 * i, 1, drain=True, prefetch=True)
        return carry

    lax.fori_loop(0, (NCH - 4) // 2, body, 0)
    proc(NCH - 2, 0, drain=True, prefetch=False)
    proc(NCH - 1, 1, drain=True, prefetch=False)
    for slot, t in ((0, NCH - 2), (1, NCH - 1)):
        scatter(t, slot, 0).wait()
        scatter(t, slot, 1).wait()


_sc_upsample = functools.partial(
    pl.kernel,
    mesh=plsc.VectorSubcoreMesh(core_axis_name="c", subcore_axis_name="s"),
    out_type=jax.ShapeDtypeStruct((B, H2, C, W2), jnp.float32),
    scratch_types=[
        pltpu.VMEM((CH, W), jnp.float32),
        pltpu.VMEM((CH, W), jnp.float32),
        pltpu.VMEM((CH, W2), jnp.float32),
        pltpu.VMEM((CH, W2), jnp.float32),
        pltpu.SemaphoreType.DMA,
        pltpu.SemaphoreType.DMA,
        pltpu.SemaphoreType.DMA,
        pltpu.SemaphoreType.DMA,
    ],
    compiler_params=pltpu.CompilerParams(needs_layout_passes=False),
)(_sc_body)


@jax.jit
def kernel(img):
    imgt = img.transpose(0, 1, 3, 2)
    outt = _sc_upsample(imgt)
    return outt.transpose(0, 1, 3, 2)
